# Initial kernel scaffold; baseline (speedup 1.0000x reference)
import jax
import jax.numpy as jnp
from jax.experimental import pallas as pl


def kernel(x, edge_index, W, b, fc_W, fc_b):
    raise NotImplementedError("write your pallas kernel here")


# speedup vs baseline: 2.0152x; 2.0152x over previous
import functools

import jax
import jax.numpy as jnp
import numpy as np
from jax import lax
from jax.experimental import pallas as pl
from jax.experimental.pallas import tpu as pltpu
from jax.experimental.pallas import tpu_sc as plsc

N = 85
NP = 96
RPT = 6
E_PAD = 1536
EPT = 96
IN_F = 128
OC = 20
OCP = 32
FR = 112
FRPT = 7
FK = 17
FO = 7
TS = 1536

def _body(xh, wh, srch, dsth, bh, tblh, fcwh, fcbh, addrh, yh,
          xv, wv, hrowv, hv, srcv, dstv, degv, degsv, dinvv, tblv,
          outpv, combv, orowv, bv, finalv, fcwv, fcbv, addrv, ybv,
          sh_h, sh_deg, sh_out, sh_final):
    t = lax.axis_index("s")
    z16f = jnp.zeros((16,), jnp.float32)
    z16i = jnp.zeros((16,), jnp.int32)
    ones_i = jnp.ones((16,), jnp.int32)
    lanes = lax.iota(jnp.int32, 16)

    pltpu.sync_copy(xh.at[pl.ds(t * RPT * IN_F, RPT * IN_F)], xv)
    pltpu.sync_copy(wh, wv)
    pltpu.sync_copy(srch.at[pl.ds(t * EPT, EPT)], srcv)
    pltpu.sync_copy(dsth.at[pl.ds(t * EPT, EPT)], dstv)
    pltpu.sync_copy(tblh, tblv)
    pltpu.sync_copy(bh, bv)
    pltpu.sync_copy(fcwh, fcwv)
    pltpu.sync_copy(fcbh, fcbv)
    pltpu.sync_copy(addrh.at[pl.ds(t * FRPT * OCP, FRPT * OCP)], addrv)

    def kblock(kb, accs):
        accs = list(accs)
        xvec = [xv[pl.ds(i * IN_F + kb * 16, 16)] for i in range(RPT)]
        for j in range(16):
            k = kb * 16 + j
            w0 = wv[pl.ds(k * OCP, 16)]
            w1 = wv[pl.ds(k * OCP + 16, 16)]
            for i in range(RPT):
                xb = xvec[i].at[jnp.full((16,), j, jnp.int32)].get(
                    mode="promise_in_bounds")
                accs[2 * i] = accs[2 * i] + xb * w0
                accs[2 * i + 1] = accs[2 * i + 1] + xb * w1
        return tuple(accs)
    accs = lax.fori_loop(0, IN_F // 16, kblock, (z16f,) * (2 * RPT))
    for i in range(RPT):
        hrowv[pl.ds(i * OCP, 16)] = accs[2 * i]
        hrowv[pl.ds(i * OCP + 16, 16)] = accs[2 * i + 1]

    for j in range(NP // 16):
        degv[pl.ds(16 * j, 16)] = z16i
    for j in range(EPT // 16):
        dvj = dstv[pl.ds(16 * j, 16)]
        for l in range(16):
            plsc.addupdate_scatter(degv, [dvj], ones_i,
                                   mask=lanes == l)

    pltpu.sync_copy(hrowv, sh_h.at[pl.ds(t * RPT * OCP, RPT * OCP)])
    pltpu.sync_copy(degv, sh_deg.at[pl.ds(t * NP, NP)])
    plsc.subcore_barrier()

    pltpu.sync_copy(sh_h, hv)
    pltpu.sync_copy(sh_deg, degsv)
    for j in range(NP // 16):
        acc = degsv[pl.ds(16 * j, 16)]
        for r in range(1, 16):
            acc = acc + degsv[pl.ds(r * NP + 16 * j, 16)]
        idx = jnp.maximum(acc - 1, 0)
        dinvv[pl.ds(16 * j, 16)] = plsc.load_gather(tblv, [idx])

    def zstep(i, c):
        outpv[pl.ds(i * 16, 16)] = z16f
        return c
    lax.fori_loop(0, NP * OCP // 16, zstep, 0)
    for j in range(EPT // 16):
        sv = srcv[pl.ds(16 * j, 16)]
        dv = dstv[pl.ds(16 * j, 16)]
        nm = plsc.load_gather(dinvv, [sv]) * plsc.load_gather(dinvv, [dv])
        for l in range(16):
            s = sv[l]
            d = dv[l]
            nr = nm[l]
            m0 = nr * hv[pl.ds(s * OCP, 16)]
            m1 = nr * hv[pl.ds(s * OCP + 16, 16)]
            outpv[pl.ds(d * OCP, 16)] = outpv[pl.ds(d * OCP, 16)] + m0
            outpv[pl.ds(d * OCP + 16, 16)] = outpv[pl.ds(d * OCP + 16, 16)] + m1

    pltpu.sync_copy(outpv, sh_out.at[pl.ds(t * NP * OCP, NP * OCP)])
    plsc.subcore_barrier()

    for r in range(16):
        pltpu.sync_copy(
            sh_out.at[pl.ds(r * NP * OCP + t * RPT * OCP, RPT * OCP)],
            combv.at[pl.ds(r * RPT * OCP, RPT * OCP)])
    for i in range(RPT):
        acc0 = bv[pl.ds(0, 16)]
        acc1 = bv[pl.ds(16, 16)]
        for r in range(16):
            acc0 = acc0 + combv[pl.ds(r * RPT * OCP + i * OCP, 16)]
            acc1 = acc1 + combv[pl.ds(r * RPT * OCP + i * OCP + 16, 16)]
        orowv[pl.ds(i * OCP, 16)] = acc0
        orowv[pl.ds(i * OCP + 16, 16)] = acc1
    pltpu.sync_copy(orowv, sh_final.at[pl.ds(t * RPT * OCP, RPT * OCP)])
    plsc.subcore_barrier()

    pltpu.sync_copy(sh_final, finalv)
    for j in range(FRPT):
        a0 = addrv[pl.ds(j * OCP, 16)]
        a1 = addrv[pl.ds(j * OCP + 16, 16)]
        v0 = jnp.maximum(plsc.load_gather(finalv, [a0]), 0.0)
        v1 = jnp.maximum(plsc.load_gather(finalv, [a1]), 0.0)
        acc = fcbv[...]
        for k in range(16):
            acc = acc + v0[k] * fcwv[pl.ds(k * 16, 16)]
        acc = acc + v1[0] * fcwv[pl.ds(16 * 16, 16)]
        ybv[pl.ds(j * 16, 16)] = acc
    pltpu.sync_copy(ybv, yh.at[pl.ds(t * FRPT * 16, FRPT * 16)])


@functools.cache
def _build_sc_kernel(interpret=False):
    return pl.kernel(
        _body,
        interpret=interpret,
        out_type=jax.ShapeDtypeStruct((FR * 16,), jnp.float32),
        mesh=plsc.VectorSubcoreMesh(
            core_axis_name="c", subcore_axis_name="s", num_cores=1,
            num_subcores=16),
        compiler_params=pltpu.CompilerParams(needs_layout_passes=False),
        scratch_types=[
            pltpu.VMEM((RPT * IN_F,), jnp.float32),
            pltpu.VMEM((IN_F * OCP,), jnp.float32),
            pltpu.VMEM((RPT * OCP,), jnp.float32),
            pltpu.VMEM((NP * OCP,), jnp.float32),
            pltpu.VMEM((EPT,), jnp.int32),
            pltpu.VMEM((EPT,), jnp.int32),
            pltpu.VMEM((NP,), jnp.int32),
            pltpu.VMEM((16 * NP,), jnp.int32),
            pltpu.VMEM((NP,), jnp.float32),
            pltpu.VMEM((TS,), jnp.float32),
            pltpu.VMEM((NP * OCP,), jnp.float32),
            pltpu.VMEM((16 * RPT * OCP,), jnp.float32),
            pltpu.VMEM((RPT * OCP,), jnp.float32),
            pltpu.VMEM((OCP,), jnp.float32),
            pltpu.VMEM((NP * OCP,), jnp.float32),
            pltpu.VMEM((OCP * 16,), jnp.float32),
            pltpu.VMEM((16,), jnp.float32),
            pltpu.VMEM((FRPT * OCP,), jnp.int32),
            pltpu.VMEM((FRPT * 16,), jnp.float32),
            pltpu.VMEM_SHARED((NP * OCP,), jnp.float32),
            pltpu.VMEM_SHARED((16 * NP,), jnp.int32),
            pltpu.VMEM_SHARED((16 * NP * OCP,), jnp.float32),
            pltpu.VMEM_SHARED((NP * OCP,), jnp.float32),
        ],
    )


_TBL = (1.0 / np.sqrt(np.arange(1, TS + 1, dtype=np.float64))).astype(np.float32)
_flat = (np.arange(FR)[:, None] * FK + np.arange(FK)[None, :])
_ADDR = ((_flat // OC) * OCP + (_flat % OC)).astype(np.int32)
_ADDR = np.pad(_ADDR, ((0, 0), (0, OCP - FK)))


def kernel(x, edge_index, W, b, fc_W, fc_b):
    ei = edge_index.astype(jnp.int32)
    n_loop = jnp.arange(N, dtype=jnp.int32)
    n_edges = ei.shape[1]
    n_pad = E_PAD - n_edges - N
    pad = jnp.full((n_pad,), N, jnp.int32)
    src = jnp.concatenate([ei[0], n_loop, pad])
    dst = jnp.concatenate([ei[1], n_loop, pad])
    xp = jnp.zeros((NP, IN_F), jnp.float32).at[:N].set(x).reshape(-1)
    wp = jnp.zeros((IN_F, OCP), jnp.float32).at[:, :OC].set(W).reshape(-1)
    bp = jnp.zeros((OCP,), jnp.float32).at[:OC].set(b)
    fcwp = jnp.zeros((OCP, 16), jnp.float32).at[:FK, :FO].set(fc_W.T).reshape(-1)
    fcbp = jnp.zeros((16,), jnp.float32).at[:FO].set(fc_b)
    yp = _build_sc_kernel()(xp, wp, src, dst, bp, jnp.asarray(_TBL), fcwp,
                            fcbp, jnp.asarray(_ADDR).reshape(-1))
    return yp.reshape(FR, 16)[:100, :FO]


# speedup vs baseline: 2.3215x; 1.1520x over previous
import functools

import jax
import jax.numpy as jnp
import numpy as np
from jax import lax
from jax.experimental import pallas as pl
from jax.experimental.pallas import tpu as pltpu
from jax.experimental.pallas import tpu_sc as plsc

N = 85
NP = 96
RPT = 6
E_PAD = 1536
EPT = 96
IN_F = 128
OC = 20
OCP = 32
FR = 112
FRPT = 7
FK = 17
FO = 7
TS = 1536

def _body(xh, wh, srch, dsth, bh, tblh, fcwh, fcbh, addrh, yh,
          xv, wv, hrowv, hv, srcv, dstv, degv, degsv, dinvv, tblv,
          outpv, combv, orowv, bv, finalv, fcwv, fcbv, addrv, ybv, sem,
          sh_h, sh_deg, sh_out, sh_final):
    t = lax.axis_index("s")
    z16f = jnp.zeros((16,), jnp.float32)
    z16i = jnp.zeros((16,), jnp.int32)
    ones_i = jnp.ones((16,), jnp.int32)
    lanes = lax.iota(jnp.int32, 16)

    descs = [
        pltpu.make_async_copy(xh.at[pl.ds(t * RPT * IN_F, RPT * IN_F)], xv, sem),
        pltpu.make_async_copy(wh, wv, sem),
        pltpu.make_async_copy(srch.at[pl.ds(t * EPT, EPT)], srcv, sem),
        pltpu.make_async_copy(dsth.at[pl.ds(t * EPT, EPT)], dstv, sem),
        pltpu.make_async_copy(tblh, tblv, sem),
        pltpu.make_async_copy(bh, bv, sem),
        pltpu.make_async_copy(fcwh, fcwv, sem),
        pltpu.make_async_copy(fcbh, fcbv, sem),
        pltpu.make_async_copy(addrh.at[pl.ds(t * FRPT * OCP, FRPT * OCP)],
                              addrv, sem),
    ]
    for de in descs:
        de.start()
    for de in descs:
        de.wait()

    def kblock(kb, accs):
        accs = list(accs)
        xvec = [xv[pl.ds(i * IN_F + kb * 16, 16)] for i in range(RPT)]
        for j in range(16):
            k = kb * 16 + j
            w0 = wv[pl.ds(k * OCP, 16)]
            w1 = wv[pl.ds(k * OCP + 16, 16)]
            for i in range(RPT):
                xb = xvec[i].at[jnp.full((16,), j, jnp.int32)].get(
                    mode="promise_in_bounds")
                accs[2 * i] = accs[2 * i] + xb * w0
                accs[2 * i + 1] = accs[2 * i + 1] + xb * w1
        return tuple(accs)
    accs = lax.fori_loop(0, IN_F // 16, kblock, (z16f,) * (2 * RPT))
    for i in range(RPT):
        hrowv[pl.ds(i * OCP, 16)] = accs[2 * i]
        hrowv[pl.ds(i * OCP + 16, 16)] = accs[2 * i + 1]

    for j in range(NP // 16):
        degv[pl.ds(16 * j, 16)] = z16i
    for j in range(EPT // 16):
        dvj = dstv[pl.ds(16 * j, 16)]
        for l in range(16):
            plsc.addupdate_scatter(degv, [dvj], ones_i,
                                   mask=lanes == l)

    descs = [
        pltpu.make_async_copy(hrowv, sh_h.at[pl.ds(t * RPT * OCP, RPT * OCP)],
                              sem),
        pltpu.make_async_copy(degv, sh_deg.at[pl.ds(t * NP, NP)], sem),
    ]
    for de in descs:
        de.start()
    for de in descs:
        de.wait()
    plsc.subcore_barrier()

    descs = [
        pltpu.make_async_copy(sh_h, hv, sem),
        pltpu.make_async_copy(sh_deg, degsv, sem),
    ]
    for de in descs:
        de.start()
    for de in descs:
        de.wait()
    for j in range(NP // 16):
        acc = degsv[pl.ds(16 * j, 16)]
        for r in range(1, 16):
            acc = acc + degsv[pl.ds(r * NP + 16 * j, 16)]
        idx = jnp.maximum(acc - 1, 0)
        dinvv[pl.ds(16 * j, 16)] = plsc.load_gather(tblv, [idx])

    def zstep(i, c):
        outpv[pl.ds(i * 16, 16)] = z16f
        return c
    lax.fori_loop(0, NP * OCP // 16, zstep, 0)
    for j in range(EPT // 16):
        sv = srcv[pl.ds(16 * j, 16)]
        dv = dstv[pl.ds(16 * j, 16)]
        nm = plsc.load_gather(dinvv, [sv]) * plsc.load_gather(dinvv, [dv])
        for l in range(16):
            s = sv[l]
            d = dv[l]
            nr = nm[l]
            m0 = nr * hv[pl.ds(s * OCP, 16)]
            m1 = nr * hv[pl.ds(s * OCP + 16, 16)]
            outpv[pl.ds(d * OCP, 16)] = outpv[pl.ds(d * OCP, 16)] + m0
            outpv[pl.ds(d * OCP + 16, 16)] = outpv[pl.ds(d * OCP + 16, 16)] + m1

    pltpu.sync_copy(outpv, sh_out.at[pl.ds(t * NP * OCP, NP * OCP)])
    plsc.subcore_barrier()

    descs = [
        pltpu.make_async_copy(
            sh_out.at[pl.ds(r * NP * OCP + t * RPT * OCP, RPT * OCP)],
            combv.at[pl.ds(r * RPT * OCP, RPT * OCP)], sem)
        for r in range(16)
    ]
    for de in descs:
        de.start()
    for de in descs:
        de.wait()
    for i in range(RPT):
        acc0 = bv[pl.ds(0, 16)]
        acc1 = bv[pl.ds(16, 16)]
        for r in range(16):
            acc0 = acc0 + combv[pl.ds(r * RPT * OCP + i * OCP, 16)]
            acc1 = acc1 + combv[pl.ds(r * RPT * OCP + i * OCP + 16, 16)]
        orowv[pl.ds(i * OCP, 16)] = acc0
        orowv[pl.ds(i * OCP + 16, 16)] = acc1
    pltpu.sync_copy(orowv, sh_final.at[pl.ds(t * RPT * OCP, RPT * OCP)])
    plsc.subcore_barrier()

    pltpu.sync_copy(sh_final, finalv)
    for j in range(FRPT):
        a0 = addrv[pl.ds(j * OCP, 16)]
        a1 = addrv[pl.ds(j * OCP + 16, 16)]
        v0 = jnp.maximum(plsc.load_gather(finalv, [a0]), 0.0)
        v1 = jnp.maximum(plsc.load_gather(finalv, [a1]), 0.0)
        acc = fcbv[...]
        for k in range(16):
            acc = acc + v0[k] * fcwv[pl.ds(k * 16, 16)]
        acc = acc + v1[0] * fcwv[pl.ds(16 * 16, 16)]
        ybv[pl.ds(j * 16, 16)] = acc
    pltpu.sync_copy(ybv, yh.at[pl.ds(t * FRPT * 16, FRPT * 16)])


@functools.cache
def _build_sc_kernel(interpret=False):
    return pl.kernel(
        _body,
        interpret=interpret,
        out_type=jax.ShapeDtypeStruct((FR * 16,), jnp.float32),
        mesh=plsc.VectorSubcoreMesh(
            core_axis_name="c", subcore_axis_name="s", num_cores=1,
            num_subcores=16),
        compiler_params=pltpu.CompilerParams(needs_layout_passes=False),
        scratch_types=[
            pltpu.VMEM((RPT * IN_F,), jnp.float32),
            pltpu.VMEM((IN_F * OCP,), jnp.float32),
            pltpu.VMEM((RPT * OCP,), jnp.float32),
            pltpu.VMEM((NP * OCP,), jnp.float32),
            pltpu.VMEM((EPT,), jnp.int32),
            pltpu.VMEM((EPT,), jnp.int32),
            pltpu.VMEM((NP,), jnp.int32),
            pltpu.VMEM((16 * NP,), jnp.int32),
            pltpu.VMEM((NP,), jnp.float32),
            pltpu.VMEM((TS,), jnp.float32),
            pltpu.VMEM((NP * OCP,), jnp.float32),
            pltpu.VMEM((16 * RPT * OCP,), jnp.float32),
            pltpu.VMEM((RPT * OCP,), jnp.float32),
            pltpu.VMEM((OCP,), jnp.float32),
            pltpu.VMEM((NP * OCP,), jnp.float32),
            pltpu.VMEM((OCP * 16,), jnp.float32),
            pltpu.VMEM((16,), jnp.float32),
            pltpu.VMEM((FRPT * OCP,), jnp.int32),
            pltpu.VMEM((FRPT * 16,), jnp.float32),
            pltpu.SemaphoreType.DMA,
            pltpu.VMEM_SHARED((NP * OCP,), jnp.float32),
            pltpu.VMEM_SHARED((16 * NP,), jnp.int32),
            pltpu.VMEM_SHARED((16 * NP * OCP,), jnp.float32),
            pltpu.VMEM_SHARED((NP * OCP,), jnp.float32),
        ],
    )


_TBL = (1.0 / np.sqrt(np.arange(1, TS + 1, dtype=np.float64))).astype(np.float32)
_flat = (np.arange(FR)[:, None] * FK + np.arange(FK)[None, :])
_ADDR = ((_flat // OC) * OCP + (_flat % OC)).astype(np.int32)
_ADDR = np.pad(_ADDR, ((0, 0), (0, OCP - FK)))


def kernel(x, edge_index, W, b, fc_W, fc_b):
    ei = edge_index.astype(jnp.int32)
    n_loop = jnp.arange(N, dtype=jnp.int32)
    n_edges = ei.shape[1]
    n_pad = E_PAD - n_edges - N
    pad = jnp.full((n_pad,), N, jnp.int32)
    src = jnp.concatenate([ei[0], n_loop, pad])
    dst = jnp.concatenate([ei[1], n_loop, pad])
    xp = jnp.zeros((NP, IN_F), jnp.float32).at[:N].set(x).reshape(-1)
    wp = jnp.zeros((IN_F, OCP), jnp.float32).at[:, :OC].set(W).reshape(-1)
    bp = jnp.zeros((OCP,), jnp.float32).at[:OC].set(b)
    fcwp = jnp.zeros((OCP, 16), jnp.float32).at[:FK, :FO].set(fc_W.T).reshape(-1)
    fcbp = jnp.zeros((16,), jnp.float32).at[:FO].set(fc_b)
    yp = _build_sc_kernel()(xp, wp, src, dst, bp, jnp.asarray(_TBL), fcwp,
                            fcbp, jnp.asarray(_ADDR).reshape(-1))
    return yp.reshape(FR, 16)[:100, :FO]


# speedup vs baseline: 2.5576x; 1.1017x over previous
import functools

import jax
import jax.numpy as jnp
import numpy as np
from jax import lax
from jax.experimental import pallas as pl
from jax.experimental.pallas import tpu as pltpu
from jax.experimental.pallas import tpu_sc as plsc

N = 85
NP = 96
RPT = 6
NE = 1360
E_PAD = 1536
EPT = 96
IN_F = 128
OC = 20
OCP = 32
FR = 112
FRPT = 7
FK = 17
FO = 7
TS = 1536


def _body(xh, eih, wh, bh, fcwh, fcbh, tblh, addrh, yh,
          xv, wv, hrowv, hv, srcv, dstv, degv, degsv, dinvv, tblv,
          outpv, combv, orowv, bv, finalv, fcwsrc, fcwv, fcbv, addrv,
          ybv, sem, sh_h, sh_deg, sh_out, sh_final):
    t = lax.axis_index("s")
    z16f = jnp.zeros((16,), jnp.float32)
    z16i = jnp.zeros((16,), jnp.int32)
    ones_i = jnp.ones((16,), jnp.int32)
    lanes = lax.iota(jnp.int32, 16)

    xoff = jnp.minimum(t * RPT * IN_F, (N - RPT) * IN_F)
    eoff = jnp.minimum(t * EPT, NE - EPT)
    descs = [
        pltpu.make_async_copy(xh.at[pl.ds(xoff, RPT * IN_F)], xv, sem),
        pltpu.make_async_copy(wh, wv.at[pl.ds(0, IN_F * OC)], sem),
        pltpu.make_async_copy(eih.at[pl.ds(eoff, EPT)], srcv, sem),
        pltpu.make_async_copy(eih.at[pl.ds(NE + eoff, EPT)], dstv, sem),
        pltpu.make_async_copy(tblh, tblv, sem),
        pltpu.make_async_copy(bh, bv.at[pl.ds(0, OC)], sem),
        pltpu.make_async_copy(fcwh, fcwsrc.at[pl.ds(0, FO * FK)], sem),
        pltpu.make_async_copy(fcbh, fcbv.at[pl.ds(0, FO)], sem),
        pltpu.make_async_copy(addrh.at[pl.ds(t * FRPT * OCP, FRPT * OCP)],
                              addrv, sem),
    ]
    for de in descs:
        de.start()
    for de in descs:
        de.wait()

    @pl.when(t == 14)
    def _t14():
        pltpu.sync_copy(xh.at[pl.ds((N - 1) * IN_F, IN_F)],
                        xv.at[pl.ds(0, IN_F)])
        for c in range(5):
            srcv[pl.ds(16 * c, 16)] = lanes + 16 * c
            dstv[pl.ds(16 * c, 16)] = lanes + 16 * c

    @pl.when(t == 15)
    def _t15():
        for c in range(6):
            v = jnp.minimum(lanes + 80 + 16 * c, N)
            srcv[pl.ds(16 * c, 16)] = v
            dstv[pl.ds(16 * c, 16)] = v

    lanes17 = lanes * FK
    for k in range(FK):
        fcwv[pl.ds(16 * k, 16)] = plsc.load_gather(fcwsrc, [lanes17 + k])

    def kblock(kb, accs):
        accs = list(accs)
        xvec = [xv[pl.ds(i * IN_F + kb * 16, 16)] for i in range(RPT)]
        for j in range(16):
            k = kb * 16 + j
            w0 = wv[pl.ds(k * OC, 16)]
            w1 = wv[pl.ds(k * OC + 16, 16)]
            for i in range(RPT):
                xb = xvec[i].at[jnp.full((16,), j, jnp.int32)].get(
                    mode="promise_in_bounds")
                accs[2 * i] = accs[2 * i] + xb * w0
                accs[2 * i + 1] = accs[2 * i + 1] + xb * w1
        return tuple(accs)
    accs = lax.fori_loop(0, IN_F // 16, kblock, (z16f,) * (2 * RPT))
    for i in range(RPT):
        hrowv[pl.ds(i * OCP, 16)] = accs[2 * i]
        hrowv[pl.ds(i * OCP + 16, 16)] = accs[2 * i + 1]

    for j in range(NP // 16):
        degv[pl.ds(16 * j, 16)] = z16i
    for j in range(EPT // 16):
        dvj = dstv[pl.ds(16 * j, 16)]
        for l in range(16):
            plsc.addupdate_scatter(degv, [dvj], ones_i,
                                   mask=lanes == l)

    descs = [
        pltpu.make_async_copy(hrowv, sh_h.at[pl.ds(t * RPT * OCP, RPT * OCP)],
                              sem),
        pltpu.make_async_copy(degv, sh_deg.at[pl.ds(t * NP, NP)], sem),
    ]
    for de in descs:
        de.start()
    for de in descs:
        de.wait()
    plsc.subcore_barrier()

    descs = [
        pltpu.make_async_copy(sh_h, hv, sem),
        pltpu.make_async_copy(sh_deg, degsv, sem),
    ]
    for de in descs:
        de.start()
    for de in descs:
        de.wait()
    for j in range(NP // 16):
        acc = degsv[pl.ds(16 * j, 16)]
        for r in range(1, 16):
            acc = acc + degsv[pl.ds(r * NP + 16 * j, 16)]
        idx = jnp.maximum(acc - 1, 0)
        dinvv[pl.ds(16 * j, 16)] = plsc.load_gather(tblv, [idx])

    def zstep(i, c):
        for u in range(4):
            outpv[pl.ds(i * 64 + u * 16, 16)] = z16f
        return c
    lax.fori_loop(0, NP * OCP // 64, zstep, 0)
    for j in range(EPT // 16):
        sv = srcv[pl.ds(16 * j, 16)]
        dv = dstv[pl.ds(16 * j, 16)]
        nm = plsc.load_gather(dinvv, [sv]) * plsc.load_gather(dinvv, [dv])
        for l in range(16):
            s = sv[l]
            d = dv[l]
            nr = nm[l]
            m0 = nr * hv[pl.ds(s * OCP, 16)]
            m1 = nr * hv[pl.ds(s * OCP + 16, 16)]
            outpv[pl.ds(d * OCP, 16)] = outpv[pl.ds(d * OCP, 16)] + m0
            outpv[pl.ds(d * OCP + 16, 16)] = outpv[pl.ds(d * OCP + 16, 16)] + m1

    pltpu.sync_copy(outpv, sh_out.at[pl.ds(t * NP * OCP, NP * OCP)])
    plsc.subcore_barrier()

    descs = [
        pltpu.make_async_copy(
            sh_out.at[pl.ds(r * NP * OCP + t * RPT * OCP, RPT * OCP)],
            combv.at[pl.ds(r * RPT * OCP, RPT * OCP)], sem)
        for r in range(16)
    ]
    for de in descs:
        de.start()
    for de in descs:
        de.wait()
    for i in range(RPT):
        acc0 = bv[pl.ds(0, 16)]
        acc1 = bv[pl.ds(16, 16)]
        for r in range(16):
            acc0 = acc0 + combv[pl.ds(r * RPT * OCP + i * OCP, 16)]
            acc1 = acc1 + combv[pl.ds(r * RPT * OCP + i * OCP + 16, 16)]
        orowv[pl.ds(i * OCP, 16)] = acc0
        orowv[pl.ds(i * OCP + 16, 16)] = acc1
    pltpu.sync_copy(orowv, sh_final.at[pl.ds(t * RPT * OCP, RPT * OCP)])
    plsc.subcore_barrier()

    pltpu.sync_copy(sh_final, finalv)
    for j in range(FRPT):
        a0 = addrv[pl.ds(j * OCP, 16)]
        a1 = addrv[pl.ds(j * OCP + 16, 16)]
        v0 = jnp.maximum(plsc.load_gather(finalv, [a0]), 0.0)
        v1 = jnp.maximum(plsc.load_gather(finalv, [a1]), 0.0)
        acc = fcbv[...]
        for k in range(16):
            acc = acc + v0[k] * fcwv[pl.ds(k * 16, 16)]
        acc = acc + v1[0] * fcwv[pl.ds(16 * 16, 16)]
        ybv[pl.ds(j * 16, 16)] = acc
    pltpu.sync_copy(ybv, yh.at[pl.ds(t * FRPT * 16, FRPT * 16)])


@functools.cache
def _build_sc_kernel(interpret=False):
    return pl.kernel(
        _body,
        interpret=interpret,
        out_type=jax.ShapeDtypeStruct((FR * 16,), jnp.float32),
        mesh=plsc.VectorSubcoreMesh(
            core_axis_name="c", subcore_axis_name="s", num_cores=1,
            num_subcores=16),
        compiler_params=pltpu.CompilerParams(needs_layout_passes=False),
        scratch_types=[
            pltpu.VMEM((RPT * IN_F,), jnp.float32),
            pltpu.VMEM((IN_F * OC + 16,), jnp.float32),
            pltpu.VMEM((RPT * OCP,), jnp.float32),
            pltpu.VMEM((NP * OCP,), jnp.float32),
            pltpu.VMEM((EPT,), jnp.int32),
            pltpu.VMEM((EPT,), jnp.int32),
            pltpu.VMEM((NP,), jnp.int32),
            pltpu.VMEM((16 * NP,), jnp.int32),
            pltpu.VMEM((NP,), jnp.float32),
            pltpu.VMEM((TS,), jnp.float32),
            pltpu.VMEM((NP * OCP,), jnp.float32),
            pltpu.VMEM((16 * RPT * OCP,), jnp.float32),
            pltpu.VMEM((RPT * OCP,), jnp.float32),
            pltpu.VMEM((OCP,), jnp.float32),
            pltpu.VMEM((NP * OCP,), jnp.float32),
            pltpu.VMEM((16 * FK,), jnp.float32),
            pltpu.VMEM((FK * 16,), jnp.float32),
            pltpu.VMEM((16,), jnp.float32),
            pltpu.VMEM((FRPT * OCP,), jnp.int32),
            pltpu.VMEM((FRPT * 16,), jnp.float32),
            pltpu.SemaphoreType.DMA,
            pltpu.VMEM_SHARED((NP * OCP,), jnp.float32),
            pltpu.VMEM_SHARED((16 * NP,), jnp.int32),
            pltpu.VMEM_SHARED((16 * NP * OCP,), jnp.float32),
            pltpu.VMEM_SHARED((NP * OCP,), jnp.float32),
        ],
    )


_TBL = (1.0 / np.sqrt(np.arange(1, TS + 1, dtype=np.float64))).astype(np.float32)
_flat = (np.arange(FR)[:, None] * FK + np.arange(FK)[None, :])
_ADDR = ((_flat // OC) * OCP + (_flat % OC)).astype(np.int32)
_ADDR = np.pad(_ADDR, ((0, 0), (0, OCP - FK)))


def kernel(x, edge_index, W, b, fc_W, fc_b):
    ei = edge_index.astype(jnp.int32).reshape(-1)
    yp = _build_sc_kernel()(
        x.reshape(-1), ei, W.reshape(-1), b, fc_W.reshape(-1), fc_b,
        jnp.asarray(_TBL), jnp.asarray(_ADDR).reshape(-1))
    return yp.reshape(FR, 16)[:100, :FO]


# speedup vs baseline: 3.0600x; 1.1964x over previous
import functools

import jax
import jax.numpy as jnp
import numpy as np
from jax import lax
from jax.experimental import pallas as pl
from jax.experimental.pallas import tpu as pltpu
from jax.experimental.pallas import tpu_sc as plsc

N = 85
NP = 96
RPT = 6
NE = 1360
E_PAD = 1536
EPT = 96
IN_F = 128
OC = 20
OCP = 32
FR = 112
FRPT = 7
FK = 17
FO = 7
TS = 1536


def _body(xh, eih, wh, bh, fcwh, fcbh, tblh, addrh, yh,
          xv, wv, hrowv, hv, srcv, dstv, degv, degsv, dinvv, tblv,
          outpv, combv, orowv, bv, finalv, fcwsrc, fcwv, fcbv, addrv,
          ybv, sem, sem2, sh_h, sh_deg, sh_out, sh_final):
    t = lax.axis_index("s")
    z16f = jnp.zeros((16,), jnp.float32)
    z16i = jnp.zeros((16,), jnp.int32)
    ones_i = jnp.ones((16,), jnp.int32)
    lanes = lax.iota(jnp.int32, 16)

    xoff = jnp.minimum(t * RPT * IN_F, (N - RPT) * IN_F)
    eoff = jnp.minimum(t * EPT, NE - EPT)
    early = [
        pltpu.make_async_copy(xh.at[pl.ds(xoff, RPT * IN_F)], xv, sem),
        pltpu.make_async_copy(wh, wv.at[pl.ds(0, IN_F * OC)], sem),
        pltpu.make_async_copy(eih.at[pl.ds(eoff, EPT)], srcv, sem),
        pltpu.make_async_copy(eih.at[pl.ds(NE + eoff, EPT)], dstv, sem),
    ]
    late = [
        pltpu.make_async_copy(tblh, tblv, sem2),
        pltpu.make_async_copy(bh, bv.at[pl.ds(0, OC)], sem2),
        pltpu.make_async_copy(fcwh, fcwsrc.at[pl.ds(0, FO * FK)], sem2),
        pltpu.make_async_copy(fcbh, fcbv.at[pl.ds(0, FO)], sem2),
        pltpu.make_async_copy(addrh.at[pl.ds(t * FRPT * OCP, FRPT * OCP)],
                              addrv, sem2),
    ]
    for de in early + late:
        de.start()
    def zstep(i, c):
        for u in range(4):
            outpv[pl.ds(i * 64 + u * 16, 16)] = z16f
        return c
    lax.fori_loop(0, NP * OCP // 64, zstep, 0)
    for j in range(NP // 16):
        degv[pl.ds(16 * j, 16)] = z16i
    for de in early:
        de.wait()

    @pl.when(t == 14)
    def _t14():
        for c in range(5):
            srcv[pl.ds(16 * c, 16)] = lanes + 16 * c
            dstv[pl.ds(16 * c, 16)] = lanes + 16 * c

    @pl.when(t == 15)
    def _t15():
        for c in range(6):
            v = jnp.minimum(lanes + 80 + 16 * c, N)
            srcv[pl.ds(16 * c, 16)] = v
            dstv[pl.ds(16 * c, 16)] = v

    def kblock(kb, accs):
        xvec = [xv[pl.ds(i * IN_F + kb * 16, 16)] for i in range(RPT)]
        def jstep(j, accs):
            accs = list(accs)
            k = kb * 16 + j
            w0 = wv[pl.ds(k * OC, 16)]
            w1 = wv[pl.ds(k * OC + 16, 16)]
            ji = jnp.full((16,), 0, jnp.int32) + j
            for i in range(RPT):
                xb = xvec[i].at[ji].get(mode="promise_in_bounds")
                accs[2 * i] = accs[2 * i] + xb * w0
                accs[2 * i + 1] = accs[2 * i + 1] + xb * w1
            return tuple(accs)
        return lax.fori_loop(0, 16, jstep, accs)
    accs = lax.fori_loop(0, IN_F // 16, kblock, (z16f,) * (2 * RPT))
    for i in range(RPT):
        hrowv[pl.ds(i * OCP, 16)] = accs[2 * i]
        hrowv[pl.ds(i * OCP + 16, 16)] = accs[2 * i + 1]

    def dstep(j, c):
        dvj = dstv[pl.ds(16 * j, 16)]
        for l in range(16):
            plsc.addupdate_scatter(degv, [dvj], ones_i,
                                   mask=lanes == l)
        return c
    lax.fori_loop(0, EPT // 16, dstep, 0)

    descs = [
        pltpu.make_async_copy(
            hrowv, sh_h.at[pl.ds(xoff // IN_F * OCP, RPT * OCP)], sem),
        pltpu.make_async_copy(degv, sh_deg.at[pl.ds(t * NP, NP)], sem),
    ]
    for de in descs:
        de.start()
    for de in descs:
        de.wait()
    plsc.subcore_barrier()

    descs = [
        pltpu.make_async_copy(sh_h, hv, sem),
        pltpu.make_async_copy(sh_deg, degsv, sem),
    ]
    for de in descs:
        de.start()
    for de in descs:
        de.wait()
    for de in late:
        de.wait()
    lanes17 = lanes * FK
    def wstep(k, c):
        fcwv[pl.ds(16 * k, 16)] = plsc.load_gather(fcwsrc, [lanes17 + k])
        return c
    lax.fori_loop(0, FK, wstep, 0)
    def gstep(j, c):
        acc = degsv[pl.ds(16 * j, 16)]
        for r in range(1, 16):
            acc = acc + degsv[pl.ds(r * NP + 16 * j, 16)]
        idx = jnp.maximum(acc - 1, 0)
        dinvv[pl.ds(16 * j, 16)] = plsc.load_gather(tblv, [idx])
        return c
    lax.fori_loop(0, NP // 16, gstep, 0)

    def estep(j, c):
        sv = srcv[pl.ds(16 * j, 16)]
        dv = dstv[pl.ds(16 * j, 16)]
        nm = plsc.load_gather(dinvv, [sv]) * plsc.load_gather(dinvv, [dv])
        for l in range(16):
            s = sv[l]
            d = dv[l]
            nr = nm[l]
            m0 = nr * hv[pl.ds(s * OCP, 16)]
            m1 = nr * hv[pl.ds(s * OCP + 16, 16)]
            outpv[pl.ds(d * OCP, 16)] = outpv[pl.ds(d * OCP, 16)] + m0
            outpv[pl.ds(d * OCP + 16, 16)] = outpv[pl.ds(d * OCP + 16, 16)] + m1
        return c
    lax.fori_loop(0, EPT // 16, estep, 0)

    pltpu.sync_copy(outpv, sh_out.at[pl.ds(t * NP * OCP, NP * OCP)])
    plsc.subcore_barrier()

    def cfire(r, c):
        pltpu.make_async_copy(
            sh_out.at[pl.ds(r * NP * OCP + t * RPT * OCP, RPT * OCP)],
            combv.at[pl.ds(r * RPT * OCP, RPT * OCP)], sem).start()
        return c
    lax.fori_loop(0, 16, cfire, 0)
    def cdrain(r, c):
        pltpu.make_async_copy(
            sh_out.at[pl.ds(r * NP * OCP + t * RPT * OCP, RPT * OCP)],
            combv.at[pl.ds(r * RPT * OCP, RPT * OCP)], sem).wait()
        return c
    lax.fori_loop(0, 16, cdrain, 0)
    def cstep(i, c):
        acc0 = bv[pl.ds(0, 16)]
        acc1 = bv[pl.ds(16, 16)]
        for r in range(16):
            acc0 = acc0 + combv[pl.ds(r * RPT * OCP + i * OCP, 16)]
            acc1 = acc1 + combv[pl.ds(r * RPT * OCP + i * OCP + 16, 16)]
        orowv[pl.ds(i * OCP, 16)] = acc0
        orowv[pl.ds(i * OCP + 16, 16)] = acc1
        return c
    lax.fori_loop(0, RPT, cstep, 0)
    pltpu.sync_copy(orowv, sh_final.at[pl.ds(t * RPT * OCP, RPT * OCP)])
    plsc.subcore_barrier()

    pltpu.sync_copy(sh_final, finalv)
    def fstep(j, c):
        a0 = addrv[pl.ds(j * OCP, 16)]
        a1 = addrv[pl.ds(j * OCP + 16, 16)]
        v0 = jnp.maximum(plsc.load_gather(finalv, [a0]), 0.0)
        v1 = jnp.maximum(plsc.load_gather(finalv, [a1]), 0.0)
        acc = fcbv[...]
        for k in range(16):
            acc = acc + v0[k] * fcwv[pl.ds(k * 16, 16)]
        acc = acc + v1[0] * fcwv[pl.ds(16 * 16, 16)]
        ybv[pl.ds(j * 16, 16)] = acc
        return c
    lax.fori_loop(0, FRPT, fstep, 0)
    pltpu.sync_copy(ybv, yh.at[pl.ds(t * FRPT * 16, FRPT * 16)])


@functools.cache
def _build_sc_kernel(interpret=False):
    return pl.kernel(
        _body,
        interpret=interpret,
        out_type=jax.ShapeDtypeStruct((FR * 16,), jnp.float32),
        mesh=plsc.VectorSubcoreMesh(
            core_axis_name="c", subcore_axis_name="s", num_cores=1,
            num_subcores=16),
        compiler_params=pltpu.CompilerParams(needs_layout_passes=False),
        scratch_types=[
            pltpu.VMEM((RPT * IN_F,), jnp.float32),
            pltpu.VMEM((IN_F * OC + 16,), jnp.float32),
            pltpu.VMEM((RPT * OCP,), jnp.float32),
            pltpu.VMEM((NP * OCP,), jnp.float32),
            pltpu.VMEM((EPT,), jnp.int32),
            pltpu.VMEM((EPT,), jnp.int32),
            pltpu.VMEM((NP,), jnp.int32),
            pltpu.VMEM((16 * NP,), jnp.int32),
            pltpu.VMEM((NP,), jnp.float32),
            pltpu.VMEM((TS,), jnp.float32),
            pltpu.VMEM((NP * OCP,), jnp.float32),
            pltpu.VMEM((16 * RPT * OCP,), jnp.float32),
            pltpu.VMEM((RPT * OCP,), jnp.float32),
            pltpu.VMEM((OCP,), jnp.float32),
            pltpu.VMEM((NP * OCP,), jnp.float32),
            pltpu.VMEM((16 * FK,), jnp.float32),
            pltpu.VMEM((FK * 16,), jnp.float32),
            pltpu.VMEM((16,), jnp.float32),
            pltpu.VMEM((FRPT * OCP,), jnp.int32),
            pltpu.VMEM((FRPT * 16,), jnp.float32),
            pltpu.SemaphoreType.DMA,
            pltpu.SemaphoreType.DMA,
            pltpu.VMEM_SHARED((NP * OCP,), jnp.float32),
            pltpu.VMEM_SHARED((16 * NP,), jnp.int32),
            pltpu.VMEM_SHARED((16 * NP * OCP,), jnp.float32),
            pltpu.VMEM_SHARED((NP * OCP,), jnp.float32),
        ],
    )


_TBL = (1.0 / np.sqrt(np.arange(1, TS + 1, dtype=np.float64))).astype(np.float32)
_flat = (np.arange(FR)[:, None] * FK + np.arange(FK)[None, :])
_ADDR = ((_flat // OC) * OCP + (_flat % OC)).astype(np.int32)
_ADDR = np.pad(_ADDR, ((0, 0), (0, OCP - FK)))


def kernel(x, edge_index, W, b, fc_W, fc_b):
    ei = edge_index.astype(jnp.int32).reshape(-1)
    yp = _build_sc_kernel()(
        x.reshape(-1), ei, W.reshape(-1), b, fc_W.reshape(-1), fc_b,
        jnp.asarray(_TBL), jnp.asarray(_ADDR).reshape(-1))
    return yp.reshape(FR, 16)[:100, :FO]
